# baseline (speedup 1.0000x reference)
import functools
import math

import jax
import jax.numpy as jnp
from jax import lax
from jax.experimental import pallas as pl
from jax.experimental.pallas import tpu as pltpu
from jax.experimental.pallas import tpu_sc as plsc

_EPS = 1e-20
_IDX_CHUNK = 128


def _sc_gather(fidx, table):
    n_rows = fidx.shape[0]
    d = table.shape[1]
    try:
        info = plsc.get_sparse_core_info()
        nc, ns = info.num_cores, info.num_subcores
    except Exception:
        nc, ns = 2, 16
    nw = nc * ns
    chunks_pw = n_rows // (nw * _IDX_CHUNK)
    rows_pw = chunks_pw * _IDX_CHUNK
    fidx3d = fidx.reshape(nw, chunks_pw, _IDX_CHUNK)
    mesh = plsc.VectorSubcoreMesh(core_axis_name="c", subcore_axis_name="s")

    @functools.partial(
        pl.kernel,
        out_type=jax.ShapeDtypeStruct((n_rows, d), jnp.float32),
        mesh=mesh,
        scratch_types=[
            pltpu.VMEM((chunks_pw, _IDX_CHUNK), jnp.int32),
            pltpu.VMEM((_IDX_CHUNK, d), jnp.float32),
            pltpu.VMEM((_IDX_CHUNK, d), jnp.float32),
            pltpu.VMEM((_IDX_CHUNK, d), jnp.float32),
            pltpu.SemaphoreType.DMA,
            pltpu.SemaphoreType.DMA,
            pltpu.SemaphoreType.DMA,
            pltpu.SemaphoreType.DMA,
            pltpu.SemaphoreType.DMA,
            pltpu.SemaphoreType.DMA,
        ],
    )
    def gather_kernel(idx_hbm, tab_hbm, out_hbm, idxv,
                      b0, b1, b2, g0, g1, g2, s0, s1, s2):
        wid = lax.axis_index("s") * nc + lax.axis_index("c")
        pltpu.sync_copy(idx_hbm.at[wid], idxv)
        bufs = (b0, b1, b2)
        gsem = (g0, g1, g2)
        ssem = (s0, s1, s2)

        def out_slice(j):
            return out_hbm.at[
                pl.ds(wid * rows_pw + j * _IDX_CHUNK, _IDX_CHUNK)]

        gather_pend = [None, None, None]
        store_pend = [None, None, None]
        for j in range(chunks_pw):
            bj = j % 3
            if store_pend[bj] is not None:
                store_pend[bj].wait()
                store_pend[bj] = None
            gather_pend[bj] = pltpu.async_copy(
                tab_hbm.at[idxv.at[j]], bufs[bj], gsem[bj])
            if j >= 1:
                p = (j - 1) % 3
                gather_pend[p].wait()
                gather_pend[p] = None
                store_pend[p] = pltpu.async_copy(
                    bufs[p], out_slice(j - 1), ssem[p])
        last = (chunks_pw - 1) % 3
        gather_pend[last].wait()
        store_pend[last] = pltpu.async_copy(
            bufs[last], out_slice(chunks_pw - 1), ssem[last])
        for sp in store_pend:
            if sp is not None:
                sp.wait()

    return gather_kernel(fidx3d, table)


def _precompute(centers):
    c, k, d = centers.shape

    def body(cent_ref, hst_ref, csq_ref):
        cent = cent_ref[...]
        sq = cent * cent
        csq_ref[...] = sq
        hst_ref[...] = 0.5 * jnp.swapaxes(jnp.sum(sq, axis=-1), 0, 1)

    return pl.pallas_call(
        body,
        out_shape=(
            jax.ShapeDtypeStruct((k, c), jnp.float32),
            jax.ShapeDtypeStruct((c, k, d), jnp.float32),
        ),
    )(centers)


def _argmax_sublanes(logits_t, k):
    iota = lax.broadcasted_iota(jnp.int32, logits_t.shape, 0)
    m = jnp.max(logits_t, axis=0, keepdims=True)
    return jnp.min(jnp.where(logits_t == m, iota, k), axis=0).astype(jnp.int32)


def _tc_iter(x2, chosen3, centers, hst, tb, off_blk, bh):
    d = x2.shape[1]
    c, k, _ = centers.shape
    grid = (bh // tb,)

    def body(x_ref, ch_ref, cent_ref, hst_ref, fidx_ref):
        x = x_ref[...]
        ch = ch_ref[...]
        tot = jnp.sum(ch, axis=0)
        for ci in range(c):
            y = x - (tot - ch[ci])
            dots_t = lax.dot_general(
                cent_ref[ci], y, (((1,), (1,)), ((), ())))
            logits_t = dots_t - hst_ref[:, ci:ci + 1]
            fidx_ref[ci, :] = _argmax_sublanes(logits_t, k) + ci * k

    return pl.pallas_call(
        body,
        grid=grid,
        in_specs=[
            pl.BlockSpec((tb, d), lambda i: (i + off_blk, 0)),
            pl.BlockSpec((c, tb, d), lambda i: (0, i, 0)),
            pl.BlockSpec((c, k, d), lambda i: (0, 0, 0)),
            pl.BlockSpec((k, c), lambda i: (0, 0)),
        ],
        out_specs=pl.BlockSpec((c, tb), lambda i: (0, i)),
        out_shape=jax.ShapeDtypeStruct((c, bh), jnp.int32),
    )(x2, chosen3, centers, hst)


def _tc_final_part(x2, chosen3, centers, csq, hst, fes, tb, off_blk, bh):
    d = x2.shape[1]
    c, k, _ = centers.shape
    grid = (bh // tb,)

    def body(x_ref, ch_ref, cent_ref, csq_ref, hst_ref, fes_ref,
             idx_ref, ap_ref, acc_ref):
        i = pl.program_id(0)

        @pl.when(i == 0)
        def _():
            ap_ref[...] = jnp.zeros_like(ap_ref)
            acc_ref[0] = 0.0
            acc_ref[1] = 0.0
            acc_ref[2] = 0.0

        x = x_ref[...]
        ch = ch_ref[...]
        tot = jnp.sum(ch, axis=0)
        scale = jnp.exp(fes_ref[0])
        mean_recon = jnp.zeros_like(x)
        fe_part = jnp.float32(0.0)
        var_part = jnp.float32(0.0)
        for ci in range(c):
            y = x - (tot - ch[ci])
            dots_t = lax.dot_general(
                cent_ref[ci], y, (((1,), (1,)), ((), ())))
            logits_t = dots_t - hst_ref[:, ci:ci + 1]
            idx_ref[ci, :] = _argmax_sublanes(logits_t, k)
            scaled = logits_t * scale
            sm = jnp.max(scaled, axis=0, keepdims=True)
            t = scaled - sm
            e = jnp.exp(t)
            s = jnp.sum(e, axis=0, keepdims=True)
            rinv = 1.0 / s
            probs = e * rinv
            fe_part += jnp.sum(
                jnp.log(s) - jnp.sum(e * t, axis=0, keepdims=True) * rinv)
            ap_ref[:, ci:ci + 1] += jnp.sum(probs, axis=1, keepdims=True)
            ec = lax.dot_general(probs, cent_ref[ci], (((0,), (0,)), ((), ())))
            mean_recon = mean_recon + ec
            vterm = lax.dot_general(probs, csq_ref[ci], (((0,), (0,)), ((), ())))
            var_part += jnp.sum(vterm - ec * ec)
        err_part = jnp.sum((x - mean_recon) ** 2)
        xsq_part = jnp.sum(x * x)
        acc_ref[0] += fe_part
        acc_ref[1] += err_part + var_part
        acc_ref[2] += xsq_part

    return pl.pallas_call(
        body,
        grid=grid,
        in_specs=[
            pl.BlockSpec((tb, d), lambda i: (i + off_blk, 0)),
            pl.BlockSpec((c, tb, d), lambda i: (0, i, 0)),
            pl.BlockSpec((c, k, d), lambda i: (0, 0, 0)),
            pl.BlockSpec((c, k, d), lambda i: (0, 0, 0)),
            pl.BlockSpec((k, c), lambda i: (0, 0)),
            pl.BlockSpec(memory_space=pltpu.SMEM),
        ],
        out_specs=(
            pl.BlockSpec((c, tb), lambda i: (0, i)),
            pl.BlockSpec((k, c), lambda i: (0, 0)),
            pl.BlockSpec(memory_space=pltpu.SMEM),
        ),
        out_shape=(
            jax.ShapeDtypeStruct((c, bh), jnp.int32),
            jax.ShapeDtypeStruct((k, c), jnp.float32),
            jax.ShapeDtypeStruct((3,), jnp.float32),
        ),
    )(x2, chosen3, centers, csq, hst, fes)


def _tc_merge(ap_a, ap_b, acc_a, acc_b, b, c, k):
    def body(apa_ref, apb_ref, acca_ref, accb_ref, el_ref, fe_ref, rl_ref):
        ap = (apa_ref[...] + apb_ref[...]) * (1.0 / b)
        ce = -jnp.sum(ap * jnp.log(ap + _EPS)) * (1.0 / c)
        el_ref[0] = math.log(k) - ce
        fe_ref[0] = (acca_ref[0] + accb_ref[0]) * (1.0 / (b * c))
        rl_ref[0] = (acca_ref[1] + accb_ref[1]) / (
            acca_ref[2] + accb_ref[2] + _EPS)

    return pl.pallas_call(
        body,
        in_specs=[
            pl.BlockSpec((k, c), lambda: (0, 0)),
            pl.BlockSpec((k, c), lambda: (0, 0)),
            pl.BlockSpec(memory_space=pltpu.SMEM),
            pl.BlockSpec(memory_space=pltpu.SMEM),
        ],
        out_specs=(
            pl.BlockSpec(memory_space=pltpu.SMEM),
            pl.BlockSpec(memory_space=pltpu.SMEM),
            pl.BlockSpec(memory_space=pltpu.SMEM),
        ),
        out_shape=(
            jax.ShapeDtypeStruct((1,), jnp.float32),
            jax.ShapeDtypeStruct((1,), jnp.float32),
            jax.ShapeDtypeStruct((1,), jnp.float32),
        ),
    )(ap_a, ap_b, acc_a, acc_b)


def kernel(x, num_iters, centers, frame_entropy_scale):
    c, k, d = centers.shape
    x2 = x.reshape(-1, d)
    b = x2.shape[0]
    tb = 512

    idx0 = jax.random.randint(jax.random.key(1), (b, c), 0, k - 1)
    cidx = jnp.arange(c, dtype=jnp.int32)
    ba = 2048
    bb = b - ba
    fidx_cb0 = idx0.T.astype(jnp.int32) + cidx[:, None] * k
    fa0 = fidx_cb0[:, :ba].reshape(c * ba)
    fb0 = fidx_cb0[:, ba:].reshape(c * bb)

    table = centers.reshape(c * k, d)
    hst, csq = _precompute(centers)

    cha0 = _sc_gather(fa0, table)

    def body(_, carry):
        cha_flat, fb = carry
        na = _tc_iter(x2, cha_flat.reshape(c, ba, d), centers, hst, tb, 0, ba)
        chb = _sc_gather(fb, table).reshape(c, bb, d)
        cha_next = _sc_gather(na.reshape(c * ba), table)
        nb = _tc_iter(x2, chb, centers, hst, tb, ba // tb, bb)
        return cha_next, nb.reshape(c * bb)

    cha_flat, fb = lax.fori_loop(0, num_iters - 1, body, (cha0, fb0))
    idx_a, ap_a, acc_a = _tc_final_part(
        x2, cha_flat.reshape(c, ba, d), centers, csq, hst,
        frame_entropy_scale, tb, 0, ba)
    chb = _sc_gather(fb, table).reshape(c, bb, d)
    idx_b, ap_b, acc_b = _tc_final_part(
        x2, chb, centers, csq, hst, frame_entropy_scale, tb, ba // tb, bb)
    el, fe, rl = _tc_merge(ap_a, ap_b, acc_a, acc_b, b, c, k)

    idx_cb = jnp.concatenate([idx_a, idx_b], axis=1)
    indexes_out = idx_cb.T.reshape(x.shape[:-1] + (c,))
    return indexes_out, el[0], fe[0], rl[0]


# speedup vs baseline: 1.0166x; 1.0166x over previous
import functools
import math

import jax
import jax.numpy as jnp
from jax import lax
from jax.experimental import pallas as pl
from jax.experimental.pallas import tpu as pltpu
from jax.experimental.pallas import tpu_sc as plsc

_EPS = 1e-20
_IDX_CHUNK = 128


def _sc_gather(fidx, table):
    n_rows = fidx.shape[0]
    d = table.shape[1]
    try:
        info = plsc.get_sparse_core_info()
        nc, ns = info.num_cores, info.num_subcores
    except Exception:
        nc, ns = 2, 16
    nw = nc * ns
    chunks_pw = n_rows // (nw * _IDX_CHUNK)
    rows_pw = chunks_pw * _IDX_CHUNK
    fidx3d = fidx.reshape(nw, chunks_pw, _IDX_CHUNK)
    mesh = plsc.VectorSubcoreMesh(core_axis_name="c", subcore_axis_name="s")

    @functools.partial(
        pl.kernel,
        out_type=jax.ShapeDtypeStruct((n_rows, d), jnp.float32),
        mesh=mesh,
        scratch_types=[
            pltpu.VMEM((chunks_pw, _IDX_CHUNK), jnp.int32),
            pltpu.VMEM((_IDX_CHUNK, d), jnp.float32),
            pltpu.VMEM((_IDX_CHUNK, d), jnp.float32),
            pltpu.SemaphoreType.DMA,
            pltpu.SemaphoreType.DMA,
            pltpu.SemaphoreType.DMA,
            pltpu.SemaphoreType.DMA,
        ],
    )
    def gather_kernel(idx_hbm, tab_hbm, out_hbm, idxv, b0, b1, g0, g1, s0, s1):
        wid = lax.axis_index("s") * nc + lax.axis_index("c")
        pltpu.sync_copy(idx_hbm.at[wid], idxv)
        bufs = (b0, b1)
        gsem = (g0, g1)
        ssem = (s0, s1)
        pending = [None, None]
        for j in range(chunks_pw):
            b = j & 1
            if pending[b] is not None:
                pending[b].wait()
            g = pltpu.async_copy(tab_hbm.at[idxv.at[j]], bufs[b], gsem[b])
            g.wait()
            s = pltpu.async_copy(
                bufs[b],
                out_hbm.at[pl.ds(wid * rows_pw + j * _IDX_CHUNK, _IDX_CHUNK)],
                ssem[b],
            )
            pending[b] = s
        for s in pending:
            if s is not None:
                s.wait()

    return gather_kernel(fidx3d, table)


def _precompute(centers):
    c, k, d = centers.shape

    def body(cent_ref, hst_ref, csq_ref):
        cent = cent_ref[...]
        sq = cent * cent
        csq_ref[...] = sq
        hst_ref[...] = 0.5 * jnp.swapaxes(jnp.sum(sq, axis=-1), 0, 1)

    return pl.pallas_call(
        body,
        out_shape=(
            jax.ShapeDtypeStruct((k, c), jnp.float32),
            jax.ShapeDtypeStruct((c, k, d), jnp.float32),
        ),
    )(centers)


def _argmax_sublanes(logits_t, k):
    iota = lax.broadcasted_iota(jnp.int32, logits_t.shape, 0)
    m = jnp.max(logits_t, axis=0, keepdims=True)
    return jnp.min(jnp.where(logits_t == m, iota, k), axis=0).astype(jnp.int32)


def _tc_iter(x2, chosen3, centers, hst, tb, off_blk, bh):
    d = x2.shape[1]
    c, k, _ = centers.shape
    grid = (bh // tb,)

    def body(x_ref, ch_ref, cent_ref, hst_ref, fidx_ref):
        x = x_ref[...]
        ch = ch_ref[...]
        tot = jnp.sum(ch, axis=0)
        for ci in range(c):
            y = x - (tot - ch[ci])
            dots_t = lax.dot_general(
                cent_ref[ci], y, (((1,), (1,)), ((), ())))
            logits_t = dots_t - hst_ref[:, ci:ci + 1]
            fidx_ref[ci, :] = _argmax_sublanes(logits_t, k) + ci * k

    return pl.pallas_call(
        body,
        grid=grid,
        in_specs=[
            pl.BlockSpec((tb, d), lambda i: (i + off_blk, 0)),
            pl.BlockSpec((c, tb, d), lambda i: (0, i, 0)),
            pl.BlockSpec((c, k, d), lambda i: (0, 0, 0)),
            pl.BlockSpec((k, c), lambda i: (0, 0)),
        ],
        out_specs=pl.BlockSpec((c, tb), lambda i: (0, i)),
        out_shape=jax.ShapeDtypeStruct((c, bh), jnp.int32),
    )(x2, chosen3, centers, hst)


def _tc_final_part(x2, chosen3, centers, csq, hst, fes, tb, off_blk, bh):
    d = x2.shape[1]
    c, k, _ = centers.shape
    grid = (bh // tb,)

    def body(x_ref, ch_ref, cent_ref, csq_ref, hst_ref, fes_ref,
             idx_ref, ap_ref, acc_ref):
        i = pl.program_id(0)

        @pl.when(i == 0)
        def _():
            ap_ref[...] = jnp.zeros_like(ap_ref)
            acc_ref[0] = 0.0
            acc_ref[1] = 0.0
            acc_ref[2] = 0.0

        x = x_ref[...]
        ch = ch_ref[...]
        tot = jnp.sum(ch, axis=0)
        scale = jnp.exp(fes_ref[0])
        mean_recon = jnp.zeros_like(x)
        fe_part = jnp.float32(0.0)
        var_part = jnp.float32(0.0)
        for ci in range(c):
            y = x - (tot - ch[ci])
            dots_t = lax.dot_general(
                cent_ref[ci], y, (((1,), (1,)), ((), ())))
            logits_t = dots_t - hst_ref[:, ci:ci + 1]
            idx_ref[ci, :] = _argmax_sublanes(logits_t, k)
            scaled = logits_t * scale
            sm = jnp.max(scaled, axis=0, keepdims=True)
            t = scaled - sm
            e = jnp.exp(t)
            s = jnp.sum(e, axis=0, keepdims=True)
            rinv = 1.0 / s
            probs = e * rinv
            fe_part += jnp.sum(
                jnp.log(s) - jnp.sum(e * t, axis=0, keepdims=True) * rinv)
            ap_ref[:, ci:ci + 1] += jnp.sum(probs, axis=1, keepdims=True)
            ec = lax.dot_general(probs, cent_ref[ci], (((0,), (0,)), ((), ())))
            mean_recon = mean_recon + ec
            vterm = lax.dot_general(probs, csq_ref[ci], (((0,), (0,)), ((), ())))
            var_part += jnp.sum(vterm - ec * ec)
        err_part = jnp.sum((x - mean_recon) ** 2)
        xsq_part = jnp.sum(x * x)
        acc_ref[0] += fe_part
        acc_ref[1] += err_part + var_part
        acc_ref[2] += xsq_part

    return pl.pallas_call(
        body,
        grid=grid,
        in_specs=[
            pl.BlockSpec((tb, d), lambda i: (i + off_blk, 0)),
            pl.BlockSpec((c, tb, d), lambda i: (0, i, 0)),
            pl.BlockSpec((c, k, d), lambda i: (0, 0, 0)),
            pl.BlockSpec((c, k, d), lambda i: (0, 0, 0)),
            pl.BlockSpec((k, c), lambda i: (0, 0)),
            pl.BlockSpec(memory_space=pltpu.SMEM),
        ],
        out_specs=(
            pl.BlockSpec((c, tb), lambda i: (0, i)),
            pl.BlockSpec((k, c), lambda i: (0, 0)),
            pl.BlockSpec(memory_space=pltpu.SMEM),
        ),
        out_shape=(
            jax.ShapeDtypeStruct((c, bh), jnp.int32),
            jax.ShapeDtypeStruct((k, c), jnp.float32),
            jax.ShapeDtypeStruct((3,), jnp.float32),
        ),
    )(x2, chosen3, centers, csq, hst, fes)


def _tc_merge(ap_a, ap_b, acc_a, acc_b, b, c, k):
    def body(apa_ref, apb_ref, acca_ref, accb_ref, el_ref, fe_ref, rl_ref):
        ap = (apa_ref[...] + apb_ref[...]) * (1.0 / b)
        ce = -jnp.sum(ap * jnp.log(ap + _EPS)) * (1.0 / c)
        el_ref[0] = math.log(k) - ce
        fe_ref[0] = (acca_ref[0] + accb_ref[0]) * (1.0 / (b * c))
        rl_ref[0] = (acca_ref[1] + accb_ref[1]) / (
            acca_ref[2] + accb_ref[2] + _EPS)

    return pl.pallas_call(
        body,
        in_specs=[
            pl.BlockSpec((k, c), lambda: (0, 0)),
            pl.BlockSpec((k, c), lambda: (0, 0)),
            pl.BlockSpec(memory_space=pltpu.SMEM),
            pl.BlockSpec(memory_space=pltpu.SMEM),
        ],
        out_specs=(
            pl.BlockSpec(memory_space=pltpu.SMEM),
            pl.BlockSpec(memory_space=pltpu.SMEM),
            pl.BlockSpec(memory_space=pltpu.SMEM),
        ),
        out_shape=(
            jax.ShapeDtypeStruct((1,), jnp.float32),
            jax.ShapeDtypeStruct((1,), jnp.float32),
            jax.ShapeDtypeStruct((1,), jnp.float32),
        ),
    )(ap_a, ap_b, acc_a, acc_b)


def kernel(x, num_iters, centers, frame_entropy_scale):
    c, k, d = centers.shape
    x2 = x.reshape(-1, d)
    b = x2.shape[0]
    tb = 512

    idx0 = jax.random.randint(jax.random.key(1), (b, c), 0, k - 1)
    cidx = jnp.arange(c, dtype=jnp.int32)
    ba = 2048
    bb = b - ba
    fidx_cb0 = idx0.T.astype(jnp.int32) + cidx[:, None] * k
    fa0 = fidx_cb0[:, :ba].reshape(c * ba)
    fb0 = fidx_cb0[:, ba:].reshape(c * bb)

    table = centers.reshape(c * k, d)
    hst, csq = _precompute(centers)

    cha0 = _sc_gather(fa0, table)

    def body(_, carry):
        cha_flat, fb = carry
        na = _tc_iter(x2, cha_flat.reshape(c, ba, d), centers, hst, tb, 0, ba)
        chb = _sc_gather(fb, table).reshape(c, bb, d)
        cha_next = _sc_gather(na.reshape(c * ba), table)
        nb = _tc_iter(x2, chb, centers, hst, tb, ba // tb, bb)
        return cha_next, nb.reshape(c * bb)

    cha_flat, fb = lax.fori_loop(0, num_iters - 1, body, (cha0, fb0))
    idx_a, ap_a, acc_a = _tc_final_part(
        x2, cha_flat.reshape(c, ba, d), centers, csq, hst,
        frame_entropy_scale, tb, 0, ba)
    chb = _sc_gather(fb, table).reshape(c, bb, d)
    idx_b, ap_b, acc_b = _tc_final_part(
        x2, chb, centers, csq, hst, frame_entropy_scale, tb, ba // tb, bb)
    el, fe, rl = _tc_merge(ap_a, ap_b, acc_a, acc_b, b, c, k)

    idx_cb = jnp.concatenate([idx_a, idx_b], axis=1)
    indexes_out = idx_cb.T.reshape(x.shape[:-1] + (c,))
    return indexes_out, el[0], fe[0], rl[0]
